# Initial kernel scaffold; baseline (speedup 1.0000x reference)
import jax
import jax.numpy as jnp
from jax.experimental import pallas as pl


def kernel(generated_img_batch, target_hist_batch):
    raise NotImplementedError("write your pallas kernel here")


# speedup vs baseline: 48.4325x; 48.4325x over previous
import functools

import jax
import jax.numpy as jnp
from jax import lax
from jax.experimental import pallas as pl
from jax.experimental.pallas import tpu as pltpu
from jax.experimental.pallas import tpu_sc as plsc

_BINS = 256
_B = 32
_PIX = 3 * 512 * 512
_NC, _NS, _L = 2, 16, 16
_CHUNK = 32768
_NCHUNK = _PIX // _CHUNK
_VECS = _CHUNK // _L

_mesh = plsc.VectorSubcoreMesh(core_axis_name="c", subcore_axis_name="s")


@functools.partial(
    pl.kernel,
    out_type=jax.ShapeDtypeStruct((_B, _L), jnp.float32),
    mesh=_mesh,
    scratch_types=[
        pltpu.VMEM((_CHUNK,), jnp.float32),
        pltpu.VMEM((_CHUNK,), jnp.float32),
        pltpu.VMEM((_BINS * _L,), jnp.float32),
        pltpu.VMEM((_BINS,), jnp.float32),
        pltpu.VMEM((_BINS,), jnp.float32),
        pltpu.VMEM((_L,), jnp.float32),
        pltpu.SemaphoreType.DMA,
        pltpu.SemaphoreType.DMA,
        pltpu.SemaphoreType.DMA,
    ],
    compiler_params=pltpu.CompilerParams(needs_layout_passes=False),
)
def _wloss(img_hbm, tgt_hbm, out_hbm, bufa, bufb, hist, tgt, btot, orow,
           sema, semb, semt):
    wid = lax.axis_index("s") * _NC + lax.axis_index("c")
    lanes = lax.iota(jnp.int32, _L)
    ones = jnp.ones((_L,), jnp.float32)
    zeros = jnp.zeros((_L,), jnp.float32)

    tcopy = pltpu.async_copy(tgt_hbm.at[wid], tgt, semt)

    def zero_body(i, _):
        hist[pl.ds(i * _L, _L)] = zeros
        return 0
    lax.fori_loop(0, _BINS, zero_body, 0)

    bufs = (bufa, bufb)
    sems = (sema, semb)
    copies = [None] * _NCHUNK
    copies[0] = pltpu.async_copy(img_hbm.at[wid, pl.ds(0, _CHUNK)], bufa, sema)
    for c in range(_NCHUNK):
        if c + 1 < _NCHUNK:
            copies[c + 1] = pltpu.async_copy(
                img_hbm.at[wid, pl.ds((c + 1) * _CHUNK, _CHUNK)],
                bufs[(c + 1) % 2], sems[(c + 1) % 2])
        copies[c].wait()
        buf = bufs[c % 2]

        def body(i, _, buf=buf):
            x = buf[pl.ds(i * _L, _L)]
            idx = (x * 127.5 + 127.5).astype(jnp.int32)
            plsc.addupdate_scatter(hist, [idx * _L + lanes], ones)
            return 0
        lax.fori_loop(0, _VECS, body, 0)

    stride = lanes * _L
    for g in range(_BINS // _L):
        gvec = zeros
        for l in range(_L):
            gvec = gvec + plsc.load_gather(hist, [stride + (g * _BINS + l)])
        btot[pl.ds(g * _L, _L)] = gvec

    tcopy.wait()

    gsv, tsv = zeros, zeros
    for g in range(_BINS // _L):
        gsv = gsv + btot[pl.ds(g * _L, _L)]
        tsv = tsv + tgt[pl.ds(g * _L, _L)]
    ginv = ones / (jnp.full((_L,), jnp.sum(gsv), jnp.float32) + 1e-8)
    tinv = ones / (jnp.full((_L,), jnp.sum(tsv), jnp.float32) + 1e-8)

    gc = jnp.float32(0.0)
    tc = jnp.float32(0.0)
    accv = zeros
    for g in range(_BINS // _L):
        gvec = btot[pl.ds(g * _L, _L)]
        tvec = tgt[pl.ds(g * _L, _L)]
        gcum = plsc.cumsum(gvec) + gc
        tcum = plsc.cumsum(tvec) + tc
        gc = gc + jnp.sum(gvec)
        tc = tc + jnp.sum(tvec)
        accv = accv + jnp.abs(gcum * ginv - tcum * tinv)

    orow[...] = accv
    pltpu.sync_copy(orow, out_hbm.at[wid])


def kernel(generated_img_batch, target_hist_batch):
    img = generated_img_batch.reshape(_B, _PIX)
    rows = _wloss(img, target_hist_batch)
    return jnp.sum(rows) / (_B * _BINS)


# speedup vs baseline: 49.9062x; 1.0304x over previous
import functools

import jax
import jax.numpy as jnp
from jax import lax
from jax.experimental import pallas as pl
from jax.experimental.pallas import tpu as pltpu
from jax.experimental.pallas import tpu_sc as plsc

_BINS = 256
_B = 32
_PIX = 3 * 512 * 512
_NC, _NS, _L = 2, 16, 16
_CHUNK = 32768
_NCHUNK = _PIX // _CHUNK
_VECS = _CHUNK // _L

_mesh = plsc.VectorSubcoreMesh(core_axis_name="c", subcore_axis_name="s")


@functools.partial(
    pl.kernel,
    out_type=jax.ShapeDtypeStruct((_B, _L), jnp.float32),
    mesh=_mesh,
    scratch_types=[
        pltpu.VMEM((_CHUNK,), jnp.float32),
        pltpu.VMEM((_CHUNK,), jnp.float32),
        pltpu.VMEM((_BINS * _L,), jnp.float32),
        pltpu.VMEM((_BINS,), jnp.float32),
        pltpu.VMEM((_BINS,), jnp.float32),
        pltpu.VMEM((_L,), jnp.float32),
        pltpu.SemaphoreType.DMA,
        pltpu.SemaphoreType.DMA,
        pltpu.SemaphoreType.DMA,
    ],
    compiler_params=pltpu.CompilerParams(needs_layout_passes=False),
)
def _wloss(img_hbm, tgt_hbm, out_hbm, bufa, bufb, hist, tgt, btot, orow,
           sema, semb, semt):
    wid = lax.axis_index("s") * _NC + lax.axis_index("c")
    lanes = lax.iota(jnp.int32, _L)
    ones = jnp.ones((_L,), jnp.float32)
    zeros = jnp.zeros((_L,), jnp.float32)

    tcopy = pltpu.async_copy(tgt_hbm.at[wid], tgt, semt)

    def zero_body(i, _):
        hist[pl.ds(i * _L, _L)] = zeros
        return 0
    lax.fori_loop(0, _BINS, zero_body, 0)

    bufs = (bufa, bufb)
    sems = (sema, semb)
    copies = [None] * _NCHUNK
    copies[0] = pltpu.async_copy(img_hbm.at[wid, pl.ds(0, _CHUNK)], bufa, sema)
    for c in range(_NCHUNK):
        if c + 1 < _NCHUNK:
            copies[c + 1] = pltpu.async_copy(
                img_hbm.at[wid, pl.ds((c + 1) * _CHUNK, _CHUNK)],
                bufs[(c + 1) % 2], sems[(c + 1) % 2])
        copies[c].wait()
        buf = bufs[c % 2]

        def body(i, _, buf=buf):
            x = buf[pl.ds(i * _L, _L)]
            idx = (x * 127.5 + 127.5).astype(jnp.int32)
            plsc.addupdate_scatter(hist, [idx * _L + lanes], ones)
            return 0
        lax.fori_loop(0, _VECS, body, 0, unroll=8)

    stride = lanes * _L
    for g in range(_BINS // _L):
        gvec = zeros
        for l in range(_L):
            gvec = gvec + plsc.load_gather(hist, [stride + (g * _BINS + l)])
        btot[pl.ds(g * _L, _L)] = gvec

    tcopy.wait()

    gsv, tsv = zeros, zeros
    for g in range(_BINS // _L):
        gsv = gsv + btot[pl.ds(g * _L, _L)]
        tsv = tsv + tgt[pl.ds(g * _L, _L)]
    ginv = ones / (jnp.full((_L,), jnp.sum(gsv), jnp.float32) + 1e-8)
    tinv = ones / (jnp.full((_L,), jnp.sum(tsv), jnp.float32) + 1e-8)

    gc = jnp.float32(0.0)
    tc = jnp.float32(0.0)
    accv = zeros
    for g in range(_BINS // _L):
        gvec = btot[pl.ds(g * _L, _L)]
        tvec = tgt[pl.ds(g * _L, _L)]
        gcum = plsc.cumsum(gvec) + gc
        tcum = plsc.cumsum(tvec) + tc
        gc = gc + jnp.sum(gvec)
        tc = tc + jnp.sum(tvec)
        accv = accv + jnp.abs(gcum * ginv - tcum * tinv)

    orow[...] = accv
    pltpu.sync_copy(orow, out_hbm.at[wid])


def kernel(generated_img_batch, target_hist_batch):
    img = generated_img_batch.reshape(_B, _PIX)
    rows = _wloss(img, target_hist_batch)
    return jnp.sum(rows) / (_B * _BINS)


# speedup vs baseline: 205.5000x; 4.1177x over previous
import functools

import jax
import jax.numpy as jnp
from jax import lax
from jax.experimental import pallas as pl
from jax.experimental.pallas import tpu as pltpu
from jax.experimental.pallas import tpu_sc as plsc

_BINS = 256
_B = 32
_PIX = 3 * 512 * 512
_NC, _NS, _L = 2, 16, 16
_CHUNK = 32768
_NCHUNK = _PIX // _CHUNK
_VECS = _CHUNK // _L

_mesh = plsc.VectorSubcoreMesh(core_axis_name="c", subcore_axis_name="s")


@functools.partial(
    pl.kernel,
    out_type=jax.ShapeDtypeStruct((_B, _L), jnp.float32),
    mesh=_mesh,
    scratch_types=[
        pltpu.VMEM((_CHUNK,), jnp.float32),
        pltpu.VMEM((_CHUNK,), jnp.float32),
        pltpu.VMEM((_BINS * _L,), jnp.float32),
        pltpu.VMEM((_BINS,), jnp.float32),
        pltpu.VMEM((_BINS,), jnp.float32),
        pltpu.VMEM((_L,), jnp.float32),
        pltpu.SemaphoreType.DMA,
        pltpu.SemaphoreType.DMA,
        pltpu.SemaphoreType.DMA,
    ],
    compiler_params=pltpu.CompilerParams(needs_layout_passes=False),
)
def _wloss(img_hbm, tgt_hbm, out_hbm, bufa, bufb, hist, tgt, btot, orow,
           sema, semb, semt):
    wid = lax.axis_index("s") * _NC + lax.axis_index("c")
    lanes = lax.iota(jnp.int32, _L)
    ones = jnp.ones((_L,), jnp.float32)
    zeros = jnp.zeros((_L,), jnp.float32)

    tcopy = pltpu.async_copy(tgt_hbm.at[wid], tgt, semt)

    def zero_body(i, _):
        hist[pl.ds(i * _L, _L)] = zeros
        return 0
    lax.fori_loop(0, _BINS, zero_body, 0)

    bufs = (bufa, bufb)
    sems = (sema, semb)
    copies = [None] * _NCHUNK
    copies[0] = pltpu.async_copy(img_hbm.at[wid, pl.ds(0, _CHUNK)], bufa, sema)
    for c in range(_NCHUNK):
        if c + 1 < _NCHUNK:
            copies[c + 1] = pltpu.async_copy(
                img_hbm.at[wid, pl.ds((c + 1) * _CHUNK, _CHUNK)],
                bufs[(c + 1) % 2], sems[(c + 1) % 2])
        copies[c].wait()
        buf = bufs[c % 2]

        @plsc.parallel_loop(0, _VECS, 1, unroll=8)
        def _(i, buf=buf):
            x = buf[pl.ds(i * _L, _L)]
            idx = (x * 127.5 + 127.5).astype(jnp.int32)
            plsc.addupdate_scatter(hist, [idx * _L + lanes], ones)

    stride = lanes * _L
    for g in range(_BINS // _L):
        gvec = zeros
        for l in range(_L):
            gvec = gvec + plsc.load_gather(hist, [stride + (g * _BINS + l)])
        btot[pl.ds(g * _L, _L)] = gvec

    tcopy.wait()

    gsv, tsv = zeros, zeros
    for g in range(_BINS // _L):
        gsv = gsv + btot[pl.ds(g * _L, _L)]
        tsv = tsv + tgt[pl.ds(g * _L, _L)]
    ginv = ones / (jnp.full((_L,), jnp.sum(gsv), jnp.float32) + 1e-8)
    tinv = ones / (jnp.full((_L,), jnp.sum(tsv), jnp.float32) + 1e-8)

    gc = jnp.float32(0.0)
    tc = jnp.float32(0.0)
    accv = zeros
    for g in range(_BINS // _L):
        gvec = btot[pl.ds(g * _L, _L)]
        tvec = tgt[pl.ds(g * _L, _L)]
        gcum = plsc.cumsum(gvec) + gc
        tcum = plsc.cumsum(tvec) + tc
        gc = gc + jnp.sum(gvec)
        tc = tc + jnp.sum(tvec)
        accv = accv + jnp.abs(gcum * ginv - tcum * tinv)

    orow[...] = accv
    pltpu.sync_copy(orow, out_hbm.at[wid])


def kernel(generated_img_batch, target_hist_batch):
    img = generated_img_batch.reshape(_B, _PIX)
    rows = _wloss(img, target_hist_batch)
    return jnp.sum(rows) / (_B * _BINS)


# speedup vs baseline: 374.8812x; 1.8242x over previous
import functools

import jax
import jax.numpy as jnp
from jax import lax
from jax.experimental import pallas as pl
from jax.experimental.pallas import tpu as pltpu
from jax.experimental.pallas import tpu_sc as plsc

_BINS = 256
_B = 32
_PIX = 3 * 512 * 512
_NC, _NS, _L = 2, 16, 16
_ROWS = 3 * 512
_CROWS = 64
_CHUNK = _CROWS * 512
_NCHUNK = _ROWS // _CROWS
_VECS = _CHUNK // _L

_mesh = plsc.VectorSubcoreMesh(core_axis_name="c", subcore_axis_name="s")


@functools.partial(
    pl.kernel,
    out_type=jax.ShapeDtypeStruct((_B, _L), jnp.float32),
    mesh=_mesh,
    scratch_types=[
        pltpu.VMEM((_CROWS, 512), jnp.float32),
        pltpu.VMEM((_CROWS, 512), jnp.float32),
        pltpu.VMEM((_BINS * _L,), jnp.float32),
        pltpu.VMEM((_BINS,), jnp.float32),
        pltpu.VMEM((_BINS,), jnp.float32),
        pltpu.VMEM((_L,), jnp.float32),
        pltpu.SemaphoreType.DMA,
        pltpu.SemaphoreType.DMA,
        pltpu.SemaphoreType.DMA,
    ],
    compiler_params=pltpu.CompilerParams(needs_layout_passes=False),
)
def _wloss(img_hbm, tgt_hbm, out_hbm, bufa, bufb, hist, tgt, btot, orow,
           sema, semb, semt):
    wid = lax.axis_index("s") * _NC + lax.axis_index("c")
    lanes = lax.iota(jnp.int32, _L)
    ones = jnp.ones((_L,), jnp.float32)
    zeros = jnp.zeros((_L,), jnp.float32)

    tcopy = pltpu.async_copy(tgt_hbm.at[wid], tgt, semt)

    def zero_body(i, _):
        hist[pl.ds(i * _L, _L)] = zeros
        return 0
    lax.fori_loop(0, _BINS, zero_body, 0)

    bufs = (bufa, bufb)
    sems = (sema, semb)
    copies = [None] * _NCHUNK
    copies[0] = pltpu.async_copy(
        img_hbm.at[wid, pl.ds(0, _CROWS), :], bufa, sema)
    for c in range(_NCHUNK):
        if c + 1 < _NCHUNK:
            copies[c + 1] = pltpu.async_copy(
                img_hbm.at[wid, pl.ds((c + 1) * _CROWS, _CROWS), :],
                bufs[(c + 1) % 2], sems[(c + 1) % 2])
        copies[c].wait()
        buf = bufs[c % 2]

        @plsc.parallel_loop(0, _VECS, 1, unroll=8)
        def _(i, buf=buf):
            r = lax.shift_right_logical(i, 5)
            col = lax.shift_left(jnp.bitwise_and(i, 31), 4)
            x = buf[r, pl.ds(col, _L)]
            idx = (x * 127.5 + 127.5).astype(jnp.int32)
            plsc.addupdate_scatter(hist, [idx * _L + lanes], ones)

    stride = lanes * _L
    for g in range(_BINS // _L):
        gvec = zeros
        for l in range(_L):
            gvec = gvec + plsc.load_gather(hist, [stride + (g * _BINS + l)])
        btot[pl.ds(g * _L, _L)] = gvec

    tcopy.wait()

    gsv, tsv = zeros, zeros
    for g in range(_BINS // _L):
        gsv = gsv + btot[pl.ds(g * _L, _L)]
        tsv = tsv + tgt[pl.ds(g * _L, _L)]
    ginv = ones / (jnp.full((_L,), jnp.sum(gsv), jnp.float32) + 1e-8)
    tinv = ones / (jnp.full((_L,), jnp.sum(tsv), jnp.float32) + 1e-8)

    gc = jnp.float32(0.0)
    tc = jnp.float32(0.0)
    accv = zeros
    for g in range(_BINS // _L):
        gvec = btot[pl.ds(g * _L, _L)]
        tvec = tgt[pl.ds(g * _L, _L)]
        gcum = plsc.cumsum(gvec) + gc
        tcum = plsc.cumsum(tvec) + tc
        gc = gc + jnp.sum(gvec)
        tc = tc + jnp.sum(tvec)
        accv = accv + jnp.abs(gcum * ginv - tcum * tinv)

    orow[...] = accv
    pltpu.sync_copy(orow, out_hbm.at[wid])


def kernel(generated_img_batch, target_hist_batch):
    img = generated_img_batch.reshape(_B, _ROWS, 512)
    rows = _wloss(img, target_hist_batch)
    return jnp.sum(rows) / (_B * _BINS)


# speedup vs baseline: 394.1137x; 1.0513x over previous
import functools

import jax
import jax.numpy as jnp
from jax import lax
from jax.experimental import pallas as pl
from jax.experimental.pallas import tpu as pltpu
from jax.experimental.pallas import tpu_sc as plsc

_BINS = 256
_B = 32
_PIX = 3 * 512 * 512
_NC, _NS, _L = 2, 16, 16
_ROWS = 3 * 512
_CROWS = 64
_CHUNK = _CROWS * 512
_NCHUNK = _ROWS // _CROWS
_VECS = _CHUNK // _L

_mesh = plsc.VectorSubcoreMesh(core_axis_name="c", subcore_axis_name="s")


@functools.partial(
    pl.kernel,
    out_type=jax.ShapeDtypeStruct((_B, _L), jnp.float32),
    mesh=_mesh,
    scratch_types=[
        pltpu.VMEM((_CROWS, 512), jnp.float32),
        pltpu.VMEM((_CROWS, 512), jnp.float32),
        pltpu.VMEM((_BINS * _L,), jnp.float32),
        pltpu.VMEM((_BINS,), jnp.float32),
        pltpu.VMEM((_BINS,), jnp.float32),
        pltpu.VMEM((_L,), jnp.float32),
        pltpu.SemaphoreType.DMA,
        pltpu.SemaphoreType.DMA,
        pltpu.SemaphoreType.DMA,
    ],
    compiler_params=pltpu.CompilerParams(needs_layout_passes=False),
)
def _wloss(img_hbm, tgt_hbm, out_hbm, bufa, bufb, hist, tgt, btot, orow,
           sema, semb, semt):
    wid = lax.axis_index("s") * _NC + lax.axis_index("c")
    lanes = lax.iota(jnp.int32, _L)
    ones = jnp.ones((_L,), jnp.float32)
    zeros = jnp.zeros((_L,), jnp.float32)

    tcopy = pltpu.async_copy(tgt_hbm.at[wid], tgt, semt)

    def zero_body(i, _):
        hist[pl.ds(i * _L, _L)] = zeros
        return 0
    lax.fori_loop(0, _BINS, zero_body, 0)

    bufs = (bufa, bufb)
    sems = (sema, semb)
    copies = [None] * _NCHUNK
    copies[0] = pltpu.async_copy(
        img_hbm.at[wid, pl.ds(0, _CROWS), :], bufa, sema)
    for c in range(_NCHUNK):
        if c + 1 < _NCHUNK:
            copies[c + 1] = pltpu.async_copy(
                img_hbm.at[wid, pl.ds((c + 1) * _CROWS, _CROWS), :],
                bufs[(c + 1) % 2], sems[(c + 1) % 2])
        copies[c].wait()
        buf = bufs[c % 2]

        @plsc.parallel_loop(0, _VECS, 1, unroll=8)
        def _(i, buf=buf):
            r = lax.shift_right_logical(i, 5)
            col = lax.shift_left(jnp.bitwise_and(i, 31), 4)
            x = buf[r, pl.ds(col, _L)]
            t = x * 127.5 + jnp.float32(8388735.0)
            bits = plsc.bitcast(t, jnp.int32)
            addr = ((bits << 4) | lanes) & 0xFFF
            plsc.addupdate_scatter(hist, [addr], ones)

    stride = lanes * _L
    for g in range(_BINS // _L):
        gvec = zeros
        for l in range(_L):
            gvec = gvec + plsc.load_gather(hist, [stride + (g * _BINS + l)])
        btot[pl.ds(g * _L, _L)] = gvec

    tcopy.wait()

    gsv, tsv = zeros, zeros
    for g in range(_BINS // _L):
        gsv = gsv + btot[pl.ds(g * _L, _L)]
        tsv = tsv + tgt[pl.ds(g * _L, _L)]
    ginv = ones / (jnp.full((_L,), jnp.sum(gsv), jnp.float32) + 1e-8)
    tinv = ones / (jnp.full((_L,), jnp.sum(tsv), jnp.float32) + 1e-8)

    gc = jnp.float32(0.0)
    tc = jnp.float32(0.0)
    accv = zeros
    for g in range(_BINS // _L):
        gvec = btot[pl.ds(g * _L, _L)]
        tvec = tgt[pl.ds(g * _L, _L)]
        gcum = plsc.cumsum(gvec) + gc
        tcum = plsc.cumsum(tvec) + tc
        gc = gc + jnp.sum(gvec)
        tc = tc + jnp.sum(tvec)
        accv = accv + jnp.abs(gcum * ginv - tcum * tinv)

    orow[...] = accv
    pltpu.sync_copy(orow, out_hbm.at[wid])


def kernel(generated_img_batch, target_hist_batch):
    img = generated_img_batch.reshape(_B, _ROWS, 512)
    rows = _wloss(img, target_hist_batch)
    return jnp.sum(rows) / (_B * _BINS)
